# baseline (speedup 1.0000x reference)
import jax
import jax.numpy as jnp
from jax.experimental import pallas as pl


def _add_kernel(x_ref, pos_ref, o_ref):
    o_ref[...] = x_ref[...] + pos_ref[...][None, :, :]


def kernel(x, pos_emb):
    batch, seq_len, dim = x.shape
    pos = pos_emb[:seq_len]
    blk = 64
    grid = (batch // blk,)
    return pl.pallas_call(
        _add_kernel,
        grid=grid,
        in_specs=[
            pl.BlockSpec((blk, seq_len, dim), lambda i: (i, 0, 0)),
            pl.BlockSpec((seq_len, dim), lambda i: (0, 0)),
        ],
        out_specs=pl.BlockSpec((blk, seq_len, dim), lambda i: (i, 0, 0)),
        out_shape=jax.ShapeDtypeStruct((batch, seq_len, dim), x.dtype),
    )(x, pos)


# speedup vs baseline: 1.6694x; 1.6694x over previous
import jax
import jax.numpy as jnp
from jax.experimental import pallas as pl


def _add_kernel(x_ref, pos_ref, o_ref):
    o_ref[...] = x_ref[...] + pos_ref[...]


def kernel(x, pos_emb):
    batch, seq_len, dim = x.shape
    flat = seq_len * dim
    x2 = x.reshape(batch, flat)
    pos = pos_emb[:seq_len].reshape(1, flat)
    blk = 128
    grid = (batch // blk,)
    out = pl.pallas_call(
        _add_kernel,
        grid=grid,
        in_specs=[
            pl.BlockSpec((blk, flat), lambda i: (i, 0)),
            pl.BlockSpec((1, flat), lambda i: (0, 0)),
        ],
        out_specs=pl.BlockSpec((blk, flat), lambda i: (i, 0)),
        out_shape=jax.ShapeDtypeStruct((batch, flat), x.dtype),
    )(x2, pos)
    return out.reshape(batch, seq_len, dim)


# speedup vs baseline: 6.3455x; 3.8011x over previous
import jax
import jax.numpy as jnp
from jax.experimental import pallas as pl


def _add_kernel(x_ref, pos_ref, o_ref):
    o_ref[...] = x_ref[...] + pos_ref[...][:, :, None]


def kernel(x, pos_emb):
    batch, seq_len, dim = x.shape
    pos = pos_emb[:seq_len]
    xt = jnp.transpose(x, (1, 2, 0))
    blk = 8
    grid = (seq_len // blk,)
    out = pl.pallas_call(
        _add_kernel,
        grid=grid,
        in_specs=[
            pl.BlockSpec((blk, dim, batch), lambda i: (i, 0, 0)),
            pl.BlockSpec((blk, dim), lambda i: (i, 0)),
        ],
        out_specs=pl.BlockSpec((blk, dim, batch), lambda i: (i, 0, 0)),
        out_shape=jax.ShapeDtypeStruct((seq_len, dim, batch), x.dtype),
    )(xt, pos)
    return jnp.transpose(out, (2, 0, 1))
